# baseline (speedup 1.0000x reference)
import functools

import jax
import jax.numpy as jnp
from jax import lax
from jax.experimental import pallas as pl
from jax.experimental.pallas import tpu as pltpu
from jax.experimental.pallas import tpu_sc as plsc

H = 1024
E = 8
TOPK = 2
I = 1408
ISH = 5632

M = 2048
A = M * TOPK
T = 256
NT = 24
NP = NT * T
BJ = 512
NJ = ISH // BJ
NJA = 5

H2 = H // 2
NW = 32
L = 16
SCH = A // L
XC = 32
YC = 32

_NEG = -1e30


def _sigmoid(x):
    return 1.0 / (1.0 + jnp.exp(-x))


def _wid():
    return lax.axis_index("s") * 2 + lax.axis_index("c")


def _router_body(x_ref, gw_ref, sgw_ref, i1_ref, i2_ref, w1_ref, w2_ref,
                 sig_ref):
    x = x_ref[...]
    gw = gw_ref[...]
    logits = lax.dot_general(x, gw, (((1,), (1,)), ((), ())),
                             preferred_element_type=jnp.float32)
    m = jnp.max(logits, axis=1, keepdims=True)
    ex = jnp.exp(logits - m)
    p = ex / jnp.sum(ex, axis=1, keepdims=True)
    iota = lax.broadcasted_iota(jnp.int32, p.shape, 1)
    m1 = jnp.max(p, axis=1, keepdims=True)
    i1 = jnp.min(jnp.where(p == m1, iota, E), axis=1, keepdims=True)
    pm = jnp.where(iota == i1, _NEG, p)
    m2 = jnp.max(pm, axis=1, keepdims=True)
    i2 = jnp.min(jnp.where(pm == m2, iota, E), axis=1, keepdims=True)
    i1_ref[...] = i1
    i2_ref[...] = i2
    w1_ref[...] = m1
    w2_ref[...] = m2
    sgw = sgw_ref[...]
    sg = lax.dot_general(x, sgw, (((1,), (1,)), ((), ())),
                         preferred_element_type=jnp.float32)
    sig_ref[...] = _sigmoid(sg)


def _dispatch_body(ids_hbm, xb_hbm, poslin_hbm, te_hbm, xperm_hbm,
                   ids_v, pos_v, te_v, sbuf_v, shpos_v,
                   idx0_v, idx1_v, rows0_v, rows1_v, ls0, ls1, ss0, ss1):
    sid = lax.axis_index("s")
    cid = lax.axis_index("c")

    @pl.when(sid == 0)
    def _():
        pltpu.sync_copy(ids_hbm, ids_v)
        lane = lax.broadcasted_iota(jnp.int32, (L,), 0)
        zero16 = jnp.zeros((L,), jnp.int32)

        def cnt(s, cs):
            v = ids_v[pl.ds(s * L, L)]
            return tuple(c + jnp.where(v == e, 1, 0)
                         for e, c in enumerate(cs))

        cs = lax.fori_loop(0, SCH, cnt, (zero16,) * E)

        sbuf_v[pl.ds(0, L)] = zero16
        po = jnp.int32(0)
        bases = []
        incls = []
        for e in range(E):
            sbuf_v[pl.ds(L, L)] = cs[e]
            pref = zero16
            for k in range(1, L):
                pref = pref + sbuf_v[pl.ds(L - k, L)]
            tot = (pref + cs[e])[L - 1]
            bases.append(pref + po)
            po = po + ((tot + T - 1) // T) * T
            incls.append(po)
        for b in range(2):
            tstart = (lane + L * b) * T
            te = zero16
            for e in range(E):
                te = te + jnp.where(incls[e] <= tstart, 1, 0)
            te_v[pl.ds(L * b, L)] = te

        def place(s, curs):
            v = ids_v[pl.ds(s * L, L)]
            pos = zero16
            out = []
            for e in range(E):
                msk = v == e
                pos = jnp.where(msk, curs[e], pos)
                out.append(curs[e] + jnp.where(msk, 1, 0))
            pos_v[pl.ds(s * L, L)] = pos
            return tuple(out)

        lax.fori_loop(0, SCH, place, tuple(bases))
        pltpu.sync_copy(pos_v, shpos_v)

        @pl.when(cid == 0)
        def _():
            pltpu.sync_copy(pos_v, poslin_hbm)
            pltpu.sync_copy(te_v, te_hbm)

    plsc.subcore_barrier()

    w = _wid()
    tw = jnp.where(w >= L, w - L, w)
    nc = 128 // XC
    idxs = (idx0_v, idx1_v)
    bufs = (rows0_v, rows1_v)
    lsems = (ls0, ls1)
    ssems = (ss0, ss1)
    loads = [None, None]
    scats = [None, None]
    pltpu.sync_copy(shpos_v.at[pl.ds(w * 128, XC)], idx0_v)
    loads[0] = pltpu.async_copy(xb_hbm.at[pl.ds(tw * 128, XC)], rows0_v, ls0)
    for c in range(nc):
        b = c % 2
        nb = (c + 1) % 2
        if c + 1 < nc:
            if scats[nb] is not None:
                scats[nb].wait()
            pltpu.sync_copy(
                shpos_v.at[pl.ds(w * 128 + (c + 1) * XC, XC)], idxs[nb])
            loads[nb] = pltpu.async_copy(
                xb_hbm.at[pl.ds(tw * 128 + (c + 1) * XC, XC)],
                bufs[nb], lsems[nb])
        loads[b].wait()
        scats[b] = pltpu.async_copy(bufs[b], xperm_hbm.at[idxs[b]],
                                    ssems[b])
    scats[0].wait()
    scats[1].wait()


def _grouped_body(te_ref, x_ref, w13g_ref, w13u_ref, w2_ref, out_ref):
    t = pl.program_id(0)

    @pl.when(te_ref[t] < E)
    def _():
        xb = x_ref[...].astype(jnp.bfloat16)
        wg = w13g_ref[0].astype(jnp.bfloat16)
        wu = w13u_ref[0].astype(jnp.bfloat16)
        g = lax.dot_general(xb, wg, (((1,), (1,)), ((), ())),
                            preferred_element_type=jnp.float32)
        u = lax.dot_general(xb, wu, (((1,), (1,)), ((), ())),
                            preferred_element_type=jnp.float32)
        h = (g * _sigmoid(g) * u).astype(jnp.bfloat16)
        w2 = w2_ref[0].astype(jnp.bfloat16)
        out_ref[...] = lax.dot_general(h, w2, (((1,), (1,)), ((), ())),
                                       preferred_element_type=jnp.float32)


def _ygather_body(yw_hbm, pos_hbm, y1_hbm, y2_hbm, idx1_v, idx2_v,
                  rows0_v, rows1_v, sem1, sem2):
    w = _wid()
    nc = 64 // YC
    pltpu.sync_copy(pos_hbm.at[pl.ds(w * 64, 64)], idx1_v)
    pltpu.sync_copy(pos_hbm.at[pl.ds(M + w * 64, 64)], idx2_v)
    for c in range(nc):
        tbase = w * 64 + c * YC
        g1 = pltpu.async_copy(yw_hbm.at[idx1_v.at[pl.ds(c * YC, YC)]],
                              rows0_v, sem1)
        g2 = pltpu.async_copy(yw_hbm.at[idx2_v.at[pl.ds(c * YC, YC)]],
                              rows1_v, sem2)
        g1.wait()
        pltpu.sync_copy(rows0_v, y1_hbm.at[pl.ds(tbase, YC)])
        g2.wait()
        pltpu.sync_copy(rows1_v, y2_hbm.at[pl.ds(tbase, YC)])


def _final_body(sha_ref, shb_ref, sig_ref, y1_ref, y2_ref, w1_ref, w2_ref,
                out_ref):
    out_ref[...] = ((sha_ref[...] + shb_ref[...]) * sig_ref[...]
                    + w1_ref[...] * y1_ref[...] + w2_ref[...] * y2_ref[...])


def _shared_body(xb_ref, wg_ref, wu_ref, wd_ref, out_ref):
    xb = xb_ref[...]
    wg = wg_ref[...].astype(jnp.bfloat16)
    wu = wu_ref[...].astype(jnp.bfloat16)
    g = lax.dot_general(xb, wg, (((1,), (1,)), ((), ())),
                        preferred_element_type=jnp.float32)
    u = lax.dot_general(xb, wu, (((1,), (1,)), ((), ())),
                        preferred_element_type=jnp.float32)
    h = (g * _sigmoid(g) * u).astype(jnp.bfloat16)
    wd = wd_ref[...].astype(jnp.bfloat16)
    y = lax.dot_general(h, wd, (((1,), (1,)), ((), ())),
                        preferred_element_type=jnp.float32)
    j = pl.program_id(0)

    @pl.when(j == 0)
    def _():
        out_ref[...] = y

    @pl.when(j > 0)
    def _():
        out_ref[...] += y


@functools.partial(jax.jit, static_argnames=("interpret",))
def _run(x32, gate_w, shared_expert_gate_w, shared_gate_up_w, shared_down_w,
         w13_stacked, w2_stacked, interpret=False):
    xb = x32.astype(jnp.bfloat16)

    i1, i2, w1, w2c, sig = pl.pallas_call(
        _router_body,
        out_shape=(jax.ShapeDtypeStruct((M, 1), jnp.int32),
                   jax.ShapeDtypeStruct((M, 1), jnp.int32),
                   jax.ShapeDtypeStruct((M, 1), jnp.float32),
                   jax.ShapeDtypeStruct((M, 1), jnp.float32),
                   jax.ShapeDtypeStruct((M, 1), jnp.float32)),
        interpret=interpret,
    )(x32, gate_w, shared_expert_gate_w)

    ids_km = jnp.concatenate([i1, i2], axis=0).reshape(A)

    sc_mesh = plsc.VectorSubcoreMesh(core_axis_name="c", subcore_axis_name="s")

    poslin, te, xperm = pl.kernel(
        _dispatch_body,
        out_type=(jax.ShapeDtypeStruct((A,), jnp.int32),
                  jax.ShapeDtypeStruct((NW,), jnp.int32),
                  jax.ShapeDtypeStruct((NP, H), jnp.float32)),
        mesh=sc_mesh,
        scratch_types=[pltpu.VMEM((A,), jnp.int32),
                       pltpu.VMEM((A,), jnp.int32),
                       pltpu.VMEM((NW,), jnp.int32),
                       pltpu.VMEM((2 * L,), jnp.int32),
                       pltpu.VMEM_SHARED((A,), jnp.int32),
                       pltpu.VMEM((XC,), jnp.int32),
                       pltpu.VMEM((XC,), jnp.int32),
                       pltpu.VMEM((XC, H), jnp.float32),
                       pltpu.VMEM((XC, H), jnp.float32),
                       pltpu.SemaphoreType.DMA,
                       pltpu.SemaphoreType.DMA,
                       pltpu.SemaphoreType.DMA,
                       pltpu.SemaphoreType.DMA],
    )(ids_km, x32)

    def shared_call(off, nj):
        return pl.pallas_call(
            _shared_body,
            grid=(nj,),
            in_specs=[
                pl.BlockSpec((M, H), lambda j: (0, 0)),
                pl.BlockSpec((BJ, H), lambda j: (j + off, 0)),
                pl.BlockSpec((BJ, H), lambda j: (j + off + NJ, 0)),
                pl.BlockSpec((H, BJ), lambda j: (0, j + off)),
            ],
            out_specs=pl.BlockSpec((M, H), lambda j: (0, 0)),
            out_shape=jax.ShapeDtypeStruct((M, H), jnp.float32),
            interpret=interpret,
        )(xb, shared_gate_up_w, shared_gate_up_w, shared_down_w)

    sha = shared_call(0, NJA)

    yw = pl.pallas_call(
        _grouped_body,
        grid_spec=pltpu.PrefetchScalarGridSpec(
            num_scalar_prefetch=1,
            grid=(NT,),
            in_specs=[
                pl.BlockSpec((T, H), lambda t, te_r: (t, 0)),
                pl.BlockSpec((1, I, H),
                             lambda t, te_r: (jnp.minimum(te_r[t], E - 1),
                                              0, 0)),
                pl.BlockSpec((1, I, H),
                             lambda t, te_r: (jnp.minimum(te_r[t], E - 1),
                                              1, 0)),
                pl.BlockSpec((1, H, I),
                             lambda t, te_r: (jnp.minimum(te_r[t], E - 1),
                                              0, 0)),
            ],
            out_specs=pl.BlockSpec((T, H), lambda t, te_r: (t, 0)),
        ),
        out_shape=jax.ShapeDtypeStruct((NP, H), jnp.float32),
        compiler_params=pltpu.CompilerParams(
            vmem_limit_bytes=63 * 1024 * 1024),
        interpret=interpret,
    )(te, xperm, w13_stacked, w13_stacked, w2_stacked)

    y1, y2 = pl.kernel(
        _ygather_body,
        out_type=(jax.ShapeDtypeStruct((M, H), jnp.float32),
                  jax.ShapeDtypeStruct((M, H), jnp.float32)),
        mesh=sc_mesh,
        scratch_types=[pltpu.VMEM((64,), jnp.int32),
                       pltpu.VMEM((64,), jnp.int32),
                       pltpu.VMEM((YC, H), jnp.float32),
                       pltpu.VMEM((YC, H), jnp.float32),
                       pltpu.SemaphoreType.DMA,
                       pltpu.SemaphoreType.DMA],
    )(yw, poslin)

    shb = shared_call(NJA, NJ - NJA)

    out = pl.pallas_call(
        _final_body,
        out_shape=jax.ShapeDtypeStruct((M, H), jnp.float32),
        interpret=interpret,
    )(sha, shb, sig, y1, y2, w1, w2c)
    return out


def kernel(hidden_states, gate_w, shared_expert_gate_w, shared_gate_up_w,
           shared_down_w, w13_stacked, w2_stacked):
    orig_shape = hidden_states.shape
    x32 = hidden_states.reshape(-1, H).astype(jnp.float32)
    out = _run(x32, gate_w, shared_expert_gate_w, shared_gate_up_w,
               shared_down_w, w13_stacked, w2_stacked)
    return out.astype(hidden_states.dtype).reshape(orig_shape)


# speedup vs baseline: 1.0889x; 1.0889x over previous
import functools

import jax
import jax.numpy as jnp
from jax import lax
from jax.experimental import pallas as pl
from jax.experimental.pallas import tpu as pltpu
from jax.experimental.pallas import tpu_sc as plsc

H = 1024
E = 8
TOPK = 2
I = 1408
ISH = 5632

M = 2048
A = M * TOPK
T = 512
NT = 16
NP = NT * T
BJ = 512
NJ = ISH // BJ

H2 = H // 2
NW = 32
L = 16
SCH = A // L
XC = 32
YC = 32

_NEG = -1e30


def _sigmoid(x):
    return 1.0 / (1.0 + jnp.exp(-x))


def _wid():
    return lax.axis_index("s") * 2 + lax.axis_index("c")


def _router_body(x_ref, gw_ref, sgw_ref, i1_ref, i2_ref, w1_ref, w2_ref,
                 sig_ref):
    x = x_ref[...]
    gw = gw_ref[...]
    logits = lax.dot_general(x, gw, (((1,), (1,)), ((), ())),
                             preferred_element_type=jnp.float32)
    m = jnp.max(logits, axis=1, keepdims=True)
    ex = jnp.exp(logits - m)
    p = ex / jnp.sum(ex, axis=1, keepdims=True)
    iota = lax.broadcasted_iota(jnp.int32, p.shape, 1)
    m1 = jnp.max(p, axis=1, keepdims=True)
    i1 = jnp.min(jnp.where(p == m1, iota, E), axis=1, keepdims=True)
    pm = jnp.where(iota == i1, _NEG, p)
    m2 = jnp.max(pm, axis=1, keepdims=True)
    i2 = jnp.min(jnp.where(pm == m2, iota, E), axis=1, keepdims=True)
    i1_ref[...] = i1
    i2_ref[...] = i2
    w1_ref[...] = m1
    w2_ref[...] = m2
    sgw = sgw_ref[...]
    sg = lax.dot_general(x, sgw, (((1,), (1,)), ((), ())),
                         preferred_element_type=jnp.float32)
    sig_ref[...] = _sigmoid(sg)


def _dispatch_body(ids_hbm, xb_hbm, poslin_hbm, te_hbm, xperm_hbm,
                   ids_v, pos_v, te_v, sbuf_v, shpos_v,
                   idx0_v, idx1_v, rows0_v, rows1_v, ls0, ls1, ss0, ss1):
    sid = lax.axis_index("s")
    cid = lax.axis_index("c")

    @pl.when(sid == 0)
    def _():
        pltpu.sync_copy(ids_hbm, ids_v)
        lane = lax.broadcasted_iota(jnp.int32, (L,), 0)
        zero16 = jnp.zeros((L,), jnp.int32)

        def cnt(s, cs):
            v = ids_v[pl.ds(s * L, L)]
            return tuple(c + jnp.where(v == e, 1, 0)
                         for e, c in enumerate(cs))

        cs = lax.fori_loop(0, SCH, cnt, (zero16,) * E)

        sbuf_v[pl.ds(0, L)] = zero16
        po = jnp.int32(0)
        bases = []
        incls = []
        for e in range(E):
            sbuf_v[pl.ds(L, L)] = cs[e]
            pref = zero16
            for k in range(1, L):
                pref = pref + sbuf_v[pl.ds(L - k, L)]
            tot = (pref + cs[e])[L - 1]
            bases.append(pref + po)
            po = po + ((tot + T - 1) // T) * T
            incls.append(po)
        for b in range(2):
            tstart = (lane + L * b) * T
            te = zero16
            for e in range(E):
                te = te + jnp.where(incls[e] <= tstart, 1, 0)
            te_v[pl.ds(L * b, L)] = te

        def place(s, curs):
            v = ids_v[pl.ds(s * L, L)]
            pos = zero16
            out = []
            for e in range(E):
                msk = v == e
                pos = jnp.where(msk, curs[e], pos)
                out.append(curs[e] + jnp.where(msk, 1, 0))
            pos_v[pl.ds(s * L, L)] = pos
            return tuple(out)

        lax.fori_loop(0, SCH, place, tuple(bases))
        pltpu.sync_copy(pos_v, shpos_v)

        @pl.when(cid == 0)
        def _():
            pltpu.sync_copy(pos_v, poslin_hbm)
            pltpu.sync_copy(te_v, te_hbm)

    plsc.subcore_barrier()

    w = _wid()
    tw = jnp.where(w >= L, w - L, w)
    nc = 128 // XC
    idxs = (idx0_v, idx1_v)
    bufs = (rows0_v, rows1_v)
    lsems = (ls0, ls1)
    ssems = (ss0, ss1)
    loads = [None, None]
    scats = [None, None]
    pltpu.sync_copy(shpos_v.at[pl.ds(w * 128, XC)], idx0_v)
    loads[0] = pltpu.async_copy(xb_hbm.at[pl.ds(tw * 128, XC)], rows0_v, ls0)
    for c in range(nc):
        b = c % 2
        nb = (c + 1) % 2
        if c + 1 < nc:
            if scats[nb] is not None:
                scats[nb].wait()
            pltpu.sync_copy(
                shpos_v.at[pl.ds(w * 128 + (c + 1) * XC, XC)], idxs[nb])
            loads[nb] = pltpu.async_copy(
                xb_hbm.at[pl.ds(tw * 128 + (c + 1) * XC, XC)],
                bufs[nb], lsems[nb])
        loads[b].wait()
        scats[b] = pltpu.async_copy(bufs[b], xperm_hbm.at[idxs[b]],
                                    ssems[b])
    scats[0].wait()
    scats[1].wait()


def _grouped_body(te_ref, x_ref, w13g_ref, w13u_ref, w2_ref, out_ref):
    t = pl.program_id(0)

    @pl.when(te_ref[t] < E)
    def _():
        xb = x_ref[...].astype(jnp.bfloat16)
        wg = w13g_ref[0].astype(jnp.bfloat16)
        wu = w13u_ref[0].astype(jnp.bfloat16)
        g = lax.dot_general(xb, wg, (((1,), (1,)), ((), ())),
                            preferred_element_type=jnp.float32)
        u = lax.dot_general(xb, wu, (((1,), (1,)), ((), ())),
                            preferred_element_type=jnp.float32)
        h = (g * _sigmoid(g) * u).astype(jnp.bfloat16)
        w2 = w2_ref[0].astype(jnp.bfloat16)
        out_ref[...] = lax.dot_general(h, w2, (((1,), (1,)), ((), ())),
                                       preferred_element_type=jnp.float32)


def _ygather_body(yw_hbm, pos_hbm, y1_hbm, y2_hbm, idx1_v, idx2_v,
                  rows0_v, rows1_v, sem1, sem2):
    w = _wid()
    nc = 64 // YC
    pltpu.sync_copy(pos_hbm.at[pl.ds(w * 64, 64)], idx1_v)
    pltpu.sync_copy(pos_hbm.at[pl.ds(M + w * 64, 64)], idx2_v)
    for c in range(nc):
        tbase = w * 64 + c * YC
        g1 = pltpu.async_copy(yw_hbm.at[idx1_v.at[pl.ds(c * YC, YC)]],
                              rows0_v, sem1)
        g2 = pltpu.async_copy(yw_hbm.at[idx2_v.at[pl.ds(c * YC, YC)]],
                              rows1_v, sem2)
        g1.wait()
        pltpu.sync_copy(rows0_v, y1_hbm.at[pl.ds(tbase, YC)])
        g2.wait()
        pltpu.sync_copy(rows1_v, y2_hbm.at[pl.ds(tbase, YC)])


def _final_body(sh_ref, sig_ref, y1_ref, y2_ref, w1_ref, w2_ref, out_ref):
    out_ref[...] = (sh_ref[...] * sig_ref[...]
                    + w1_ref[...] * y1_ref[...] + w2_ref[...] * y2_ref[...])


def _shared_body(xb_ref, wg_ref, wu_ref, wd_ref, out_ref):
    xb = xb_ref[...]
    wg = wg_ref[...].astype(jnp.bfloat16)
    wu = wu_ref[...].astype(jnp.bfloat16)
    g = lax.dot_general(xb, wg, (((1,), (1,)), ((), ())),
                        preferred_element_type=jnp.float32)
    u = lax.dot_general(xb, wu, (((1,), (1,)), ((), ())),
                        preferred_element_type=jnp.float32)
    h = (g * _sigmoid(g) * u).astype(jnp.bfloat16)
    wd = wd_ref[...].astype(jnp.bfloat16)
    y = lax.dot_general(h, wd, (((1,), (1,)), ((), ())),
                        preferred_element_type=jnp.float32)
    j = pl.program_id(0)

    @pl.when(j == 0)
    def _():
        out_ref[...] = y

    @pl.when(j > 0)
    def _():
        out_ref[...] += y


@functools.partial(jax.jit, static_argnames=("interpret",))
def _run(x32, gate_w, shared_expert_gate_w, shared_gate_up_w, shared_down_w,
         w13_stacked, w2_stacked, interpret=False):
    xb = x32.astype(jnp.bfloat16)

    i1, i2, w1, w2c, sig = pl.pallas_call(
        _router_body,
        out_shape=(jax.ShapeDtypeStruct((M, 1), jnp.int32),
                   jax.ShapeDtypeStruct((M, 1), jnp.int32),
                   jax.ShapeDtypeStruct((M, 1), jnp.float32),
                   jax.ShapeDtypeStruct((M, 1), jnp.float32),
                   jax.ShapeDtypeStruct((M, 1), jnp.float32)),
        interpret=interpret,
    )(x32, gate_w, shared_expert_gate_w)

    ids_km = jnp.concatenate([i1, i2], axis=0).reshape(A)

    sc_mesh = plsc.VectorSubcoreMesh(core_axis_name="c", subcore_axis_name="s")

    poslin, te, xperm = pl.kernel(
        _dispatch_body,
        out_type=(jax.ShapeDtypeStruct((A,), jnp.int32),
                  jax.ShapeDtypeStruct((NW,), jnp.int32),
                  jax.ShapeDtypeStruct((NP, H), jnp.float32)),
        mesh=sc_mesh,
        scratch_types=[pltpu.VMEM((A,), jnp.int32),
                       pltpu.VMEM((A,), jnp.int32),
                       pltpu.VMEM((NW,), jnp.int32),
                       pltpu.VMEM((2 * L,), jnp.int32),
                       pltpu.VMEM_SHARED((A,), jnp.int32),
                       pltpu.VMEM((XC,), jnp.int32),
                       pltpu.VMEM((XC,), jnp.int32),
                       pltpu.VMEM((XC, H), jnp.float32),
                       pltpu.VMEM((XC, H), jnp.float32),
                       pltpu.SemaphoreType.DMA,
                       pltpu.SemaphoreType.DMA,
                       pltpu.SemaphoreType.DMA,
                       pltpu.SemaphoreType.DMA],
    )(ids_km, x32)

    yw = pl.pallas_call(
        _grouped_body,
        grid_spec=pltpu.PrefetchScalarGridSpec(
            num_scalar_prefetch=1,
            grid=(NT,),
            in_specs=[
                pl.BlockSpec((T, H), lambda t, te_r: (t, 0)),
                pl.BlockSpec((1, I, H),
                             lambda t, te_r: (jnp.minimum(te_r[t], E - 1),
                                              0, 0)),
                pl.BlockSpec((1, I, H),
                             lambda t, te_r: (jnp.minimum(te_r[t], E - 1),
                                              1, 0)),
                pl.BlockSpec((1, H, I),
                             lambda t, te_r: (jnp.minimum(te_r[t], E - 1),
                                              0, 0)),
            ],
            out_specs=pl.BlockSpec((T, H), lambda t, te_r: (t, 0)),
        ),
        out_shape=jax.ShapeDtypeStruct((NP, H), jnp.float32),
        compiler_params=pltpu.CompilerParams(
            vmem_limit_bytes=63 * 1024 * 1024),
        interpret=interpret,
    )(te, xperm, w13_stacked, w13_stacked, w2_stacked)

    sh = pl.pallas_call(
        _shared_body,
        grid=(NJ,),
        in_specs=[
            pl.BlockSpec((M, H), lambda j: (0, 0)),
            pl.BlockSpec((BJ, H), lambda j: (j, 0)),
            pl.BlockSpec((BJ, H), lambda j: (j + NJ, 0)),
            pl.BlockSpec((H, BJ), lambda j: (0, j)),
        ],
        out_specs=pl.BlockSpec((M, H), lambda j: (0, 0)),
        out_shape=jax.ShapeDtypeStruct((M, H), jnp.float32),
        interpret=interpret,
    )(xb, shared_gate_up_w, shared_gate_up_w, shared_down_w)

    y1, y2 = pl.kernel(
        _ygather_body,
        out_type=(jax.ShapeDtypeStruct((M, H), jnp.float32),
                  jax.ShapeDtypeStruct((M, H), jnp.float32)),
        mesh=sc_mesh,
        scratch_types=[pltpu.VMEM((64,), jnp.int32),
                       pltpu.VMEM((64,), jnp.int32),
                       pltpu.VMEM((YC, H), jnp.float32),
                       pltpu.VMEM((YC, H), jnp.float32),
                       pltpu.SemaphoreType.DMA,
                       pltpu.SemaphoreType.DMA],
    )(yw, poslin)

    out = pl.pallas_call(
        _final_body,
        out_shape=jax.ShapeDtypeStruct((M, H), jnp.float32),
        interpret=interpret,
    )(sh, sig, y1, y2, w1, w2c)
    return out


def kernel(hidden_states, gate_w, shared_expert_gate_w, shared_gate_up_w,
           shared_down_w, w13_stacked, w2_stacked):
    orig_shape = hidden_states.shape
    x32 = hidden_states.reshape(-1, H).astype(jnp.float32)
    out = _run(x32, gate_w, shared_expert_gate_w, shared_gate_up_w,
               shared_down_w, w13_stacked, w2_stacked)
    return out.astype(hidden_states.dtype).reshape(orig_shape)


# speedup vs baseline: 1.0935x; 1.0042x over previous
import functools

import jax
import jax.numpy as jnp
from jax import lax
from jax.experimental import pallas as pl
from jax.experimental.pallas import tpu as pltpu
from jax.experimental.pallas import tpu_sc as plsc

H = 1024
E = 8
TOPK = 2
I = 1408
ISH = 5632

M = 2048
A = M * TOPK
T = 512
NT = 16
NP = NT * T
BJ = 512
NJ = ISH // BJ

H2 = H // 2
NW = 32
L = 16
SCH = A // L
XC = 32
YC = 32

_NEG = -1e30


def _sigmoid(x):
    return 1.0 / (1.0 + jnp.exp(-x))


def _wid():
    return lax.axis_index("s") * 2 + lax.axis_index("c")


def _router_body(x_ref, gw_ref, sgw_ref, i1_ref, i2_ref, w1_ref, w2_ref,
                 sig_ref):
    x = x_ref[...]
    gw = gw_ref[...]
    logits = lax.dot_general(x, gw, (((1,), (1,)), ((), ())),
                             preferred_element_type=jnp.float32)
    m = jnp.max(logits, axis=1, keepdims=True)
    ex = jnp.exp(logits - m)
    p = ex / jnp.sum(ex, axis=1, keepdims=True)
    iota = lax.broadcasted_iota(jnp.int32, p.shape, 1)
    m1 = jnp.max(p, axis=1, keepdims=True)
    i1 = jnp.min(jnp.where(p == m1, iota, E), axis=1, keepdims=True)
    pm = jnp.where(iota == i1, _NEG, p)
    m2 = jnp.max(pm, axis=1, keepdims=True)
    i2 = jnp.min(jnp.where(pm == m2, iota, E), axis=1, keepdims=True)
    i1_ref[...] = i1
    i2_ref[...] = i2
    w1_ref[...] = m1
    w2_ref[...] = m2
    sgw = sgw_ref[...]
    sg = lax.dot_general(x, sgw, (((1,), (1,)), ((), ())),
                         preferred_element_type=jnp.float32)
    sig_ref[...] = _sigmoid(sg)


def _dispatch_body(ids_hbm, xb_hbm, poslin_hbm, te_hbm, xperm_hbm,
                   ids_v, pos_v, te_v, sbuf_v, shpos_v,
                   idx0_v, idx1_v, rows0_v, rows1_v, ls0, ls1, ss0, ss1):
    sid = lax.axis_index("s")
    cid = lax.axis_index("c")

    @pl.when(sid == 0)
    def _():
        pltpu.sync_copy(ids_hbm, ids_v)
        lane = lax.broadcasted_iota(jnp.int32, (L,), 0)
        zero16 = jnp.zeros((L,), jnp.int32)

        def cnt(s, cs):
            v = ids_v[pl.ds(s * L, L)]
            return tuple(c + jnp.where(v == e, 1, 0)
                         for e, c in enumerate(cs))

        cs = lax.fori_loop(0, SCH, cnt, (zero16,) * E)

        sbuf_v[pl.ds(0, L)] = zero16
        po = jnp.int32(0)
        bases = []
        incls = []
        for e in range(E):
            sbuf_v[pl.ds(L, L)] = cs[e]
            pref = zero16
            for k in range(1, L):
                pref = pref + sbuf_v[pl.ds(L - k, L)]
            tot = (pref + cs[e])[L - 1]
            bases.append(pref + po)
            po = po + ((tot + T - 1) // T) * T
            incls.append(po)
        for b in range(2):
            tstart = (lane + L * b) * T
            te = zero16
            for e in range(E):
                te = te + jnp.where(incls[e] <= tstart, 1, 0)
            te_v[pl.ds(L * b, L)] = te

        def place(s, curs):
            v = ids_v[pl.ds(s * L, L)]
            pos = zero16
            out = []
            for e in range(E):
                msk = v == e
                pos = jnp.where(msk, curs[e], pos)
                out.append(curs[e] + jnp.where(msk, 1, 0))
            pos_v[pl.ds(s * L, L)] = pos
            return tuple(out)

        lax.fori_loop(0, SCH, place, tuple(bases))
        pltpu.sync_copy(pos_v, shpos_v)

        @pl.when(cid == 0)
        def _():
            pltpu.sync_copy(pos_v, poslin_hbm)
            pltpu.sync_copy(te_v, te_hbm)

    plsc.subcore_barrier()

    w = _wid()
    tw = jnp.where(w >= L, w - L, w)
    nc = 128 // XC
    idxs = (idx0_v, idx1_v)
    bufs = (rows0_v, rows1_v)
    lsems = (ls0, ls1)
    ssems = (ss0, ss1)
    loads = [None, None]
    scats = [None, None]
    pltpu.sync_copy(shpos_v.at[pl.ds(w * 128, XC)], idx0_v)
    loads[0] = pltpu.async_copy(xb_hbm.at[pl.ds(tw * 128, XC)], rows0_v, ls0)
    for c in range(nc):
        b = c % 2
        nb = (c + 1) % 2
        if c + 1 < nc:
            if scats[nb] is not None:
                scats[nb].wait()
            pltpu.sync_copy(
                shpos_v.at[pl.ds(w * 128 + (c + 1) * XC, XC)], idxs[nb])
            loads[nb] = pltpu.async_copy(
                xb_hbm.at[pl.ds(tw * 128 + (c + 1) * XC, XC)],
                bufs[nb], lsems[nb])
        loads[b].wait()
        scats[b] = pltpu.async_copy(bufs[b], xperm_hbm.at[idxs[b]],
                                    ssems[b])
    scats[0].wait()
    scats[1].wait()


def _grouped_body(te_ref, x_ref, w13g_ref, w13u_ref, w2_ref, out_ref):
    t = pl.program_id(0)

    @pl.when(te_ref[t] < E)
    def _():
        xb = x_ref[...].astype(jnp.bfloat16)
        wg = w13g_ref[0].astype(jnp.bfloat16)
        wu = w13u_ref[0].astype(jnp.bfloat16)
        g = lax.dot_general(xb, wg, (((1,), (1,)), ((), ())),
                            preferred_element_type=jnp.float32)
        u = lax.dot_general(xb, wu, (((1,), (1,)), ((), ())),
                            preferred_element_type=jnp.float32)
        h = (g * _sigmoid(g) * u).astype(jnp.bfloat16)
        w2 = w2_ref[0].astype(jnp.bfloat16)
        out_ref[...] = lax.dot_general(h, w2, (((1,), (1,)), ((), ())),
                                       preferred_element_type=jnp.float32)


def _ygather_body(yw_hbm, pos_hbm, y1_hbm, y2_hbm, idx1_v, idx2_v,
                  rows0_v, rows1_v, sem1, sem2):
    w = _wid()
    nc = 64 // YC
    pltpu.sync_copy(pos_hbm.at[pl.ds(w * 64, 64)], idx1_v)
    pltpu.sync_copy(pos_hbm.at[pl.ds(M + w * 64, 64)], idx2_v)
    for c in range(nc):
        tbase = w * 64 + c * YC
        g1 = pltpu.async_copy(yw_hbm.at[idx1_v.at[pl.ds(c * YC, YC)]],
                              rows0_v, sem1)
        g2 = pltpu.async_copy(yw_hbm.at[idx2_v.at[pl.ds(c * YC, YC)]],
                              rows1_v, sem2)
        g1.wait()
        pltpu.sync_copy(rows0_v, y1_hbm.at[pl.ds(tbase, YC)])
        g2.wait()
        pltpu.sync_copy(rows1_v, y2_hbm.at[pl.ds(tbase, YC)])


def _final_body(sh_ref, sig_ref, y1_ref, y2_ref, w1_ref, w2_ref, out_ref):
    out_ref[...] = (sh_ref[...] * sig_ref[...]
                    + w1_ref[...] * y1_ref[...] + w2_ref[...] * y2_ref[...])


def _shared_body(xb_ref, wg_ref, wu_ref, wd_ref, out_ref):
    xb = xb_ref[...]
    wg = wg_ref[...].astype(jnp.bfloat16)
    wu = wu_ref[...].astype(jnp.bfloat16)
    g = lax.dot_general(xb, wg, (((1,), (1,)), ((), ())),
                        preferred_element_type=jnp.float32)
    u = lax.dot_general(xb, wu, (((1,), (1,)), ((), ())),
                        preferred_element_type=jnp.float32)
    h = (g * _sigmoid(g) * u).astype(jnp.bfloat16)
    wd = wd_ref[...].astype(jnp.bfloat16)
    y = lax.dot_general(h, wd, (((1,), (1,)), ((), ())),
                        preferred_element_type=jnp.float32)
    j = pl.program_id(0)

    @pl.when(j == 0)
    def _():
        out_ref[...] = y

    @pl.when(j > 0)
    def _():
        out_ref[...] += y


@functools.partial(jax.jit, static_argnames=("interpret",))
def _run(x32, gate_w, shared_expert_gate_w, shared_gate_up_w, shared_down_w,
         w13_stacked, w2_stacked, interpret=False):
    xb = x32.astype(jnp.bfloat16)

    i1, i2, w1, w2c, sig = pl.pallas_call(
        _router_body,
        out_shape=(jax.ShapeDtypeStruct((M, 1), jnp.int32),
                   jax.ShapeDtypeStruct((M, 1), jnp.int32),
                   jax.ShapeDtypeStruct((M, 1), jnp.float32),
                   jax.ShapeDtypeStruct((M, 1), jnp.float32),
                   jax.ShapeDtypeStruct((M, 1), jnp.float32)),
        interpret=interpret,
    )(x32, gate_w, shared_expert_gate_w)

    ids_km = jnp.concatenate([i1, i2], axis=0).reshape(A)

    sc_mesh = plsc.VectorSubcoreMesh(core_axis_name="c", subcore_axis_name="s")

    poslin, te, xperm = pl.kernel(
        _dispatch_body,
        out_type=(jax.ShapeDtypeStruct((A,), jnp.int32),
                  jax.ShapeDtypeStruct((NW,), jnp.int32),
                  jax.ShapeDtypeStruct((NP, H), jnp.float32)),
        mesh=sc_mesh,
        scratch_types=[pltpu.VMEM((A,), jnp.int32),
                       pltpu.VMEM((A,), jnp.int32),
                       pltpu.VMEM((NW,), jnp.int32),
                       pltpu.VMEM((2 * L,), jnp.int32),
                       pltpu.VMEM_SHARED((A,), jnp.int32),
                       pltpu.VMEM((XC,), jnp.int32),
                       pltpu.VMEM((XC,), jnp.int32),
                       pltpu.VMEM((XC, H), jnp.float32),
                       pltpu.VMEM((XC, H), jnp.float32),
                       pltpu.SemaphoreType.DMA,
                       pltpu.SemaphoreType.DMA,
                       pltpu.SemaphoreType.DMA,
                       pltpu.SemaphoreType.DMA],
    )(ids_km, x32)

    yw = pl.pallas_call(
        _grouped_body,
        grid_spec=pltpu.PrefetchScalarGridSpec(
            num_scalar_prefetch=1,
            grid=(NT,),
            in_specs=[
                pl.BlockSpec((T, H), lambda t, te_r: (t, 0)),
                pl.BlockSpec((1, I, H),
                             lambda t, te_r: (jnp.minimum(te_r[t], E - 1),
                                              0, 0)),
                pl.BlockSpec((1, I, H),
                             lambda t, te_r: (jnp.minimum(te_r[t], E - 1),
                                              1, 0)),
                pl.BlockSpec((1, H, I),
                             lambda t, te_r: (jnp.minimum(te_r[t], E - 1),
                                              0, 0)),
            ],
            out_specs=pl.BlockSpec((T, H), lambda t, te_r: (t, 0)),
        ),
        out_shape=jax.ShapeDtypeStruct((NP, H), jnp.float32),
        compiler_params=pltpu.CompilerParams(
            vmem_limit_bytes=63 * 1024 * 1024),
        interpret=interpret,
    )(te, xperm, w13_stacked, w13_stacked, w2_stacked)

    y1, y2 = pl.kernel(
        _ygather_body,
        out_type=(jax.ShapeDtypeStruct((M, H), jnp.float32),
                  jax.ShapeDtypeStruct((M, H), jnp.float32)),
        mesh=sc_mesh,
        scratch_types=[pltpu.VMEM((64,), jnp.int32),
                       pltpu.VMEM((64,), jnp.int32),
                       pltpu.VMEM((YC, H), jnp.float32),
                       pltpu.VMEM((YC, H), jnp.float32),
                       pltpu.SemaphoreType.DMA,
                       pltpu.SemaphoreType.DMA],
    )(yw, poslin)

    sh = pl.pallas_call(
        _shared_body,
        grid=(NJ,),
        in_specs=[
            pl.BlockSpec((M, H), lambda j: (0, 0)),
            pl.BlockSpec((BJ, H), lambda j: (j, 0)),
            pl.BlockSpec((BJ, H), lambda j: (j + NJ, 0)),
            pl.BlockSpec((H, BJ), lambda j: (0, j)),
        ],
        out_specs=pl.BlockSpec((M, H), lambda j: (0, 0)),
        out_shape=jax.ShapeDtypeStruct((M, H), jnp.float32),
        interpret=interpret,
    )(xb, shared_gate_up_w, shared_gate_up_w, shared_down_w)

    out = pl.pallas_call(
        _final_body,
        out_shape=jax.ShapeDtypeStruct((M, H), jnp.float32),
        interpret=interpret,
    )(sh, sig, y1, y2, w1, w2c)
    return out


def kernel(hidden_states, gate_w, shared_expert_gate_w, shared_gate_up_w,
           shared_down_w, w13_stacked, w2_stacked):
    orig_shape = hidden_states.shape
    x32 = hidden_states.reshape(-1, H).astype(jnp.float32)
    out = _run(x32, gate_w, shared_expert_gate_w, shared_gate_up_w,
               shared_down_w, w13_stacked, w2_stacked)
    return out.astype(hidden_states.dtype).reshape(orig_shape)


# speedup vs baseline: 1.1149x; 1.0196x over previous
import functools

import jax
import jax.numpy as jnp
from jax import lax
from jax.experimental import pallas as pl
from jax.experimental.pallas import tpu as pltpu
from jax.experimental.pallas import tpu_sc as plsc

H = 1024
E = 8
TOPK = 2
I = 1408
ISH = 5632

M = 2048
A = M * TOPK
T = 640
NT = 15
NP = NT * T
BJ = 512
NJ = ISH // BJ

H2 = H // 2
NW = 32
L = 16
SCH = A // L
XC = 32
YC = 32

_NEG = -1e30


def _sigmoid(x):
    return 1.0 / (1.0 + jnp.exp(-x))


def _wid():
    return lax.axis_index("s") * 2 + lax.axis_index("c")


def _router_body(x_ref, gw_ref, sgw_ref, i1_ref, i2_ref, w1_ref, w2_ref,
                 sig_ref):
    x = x_ref[...]
    gw = gw_ref[...]
    logits = lax.dot_general(x, gw, (((1,), (1,)), ((), ())),
                             preferred_element_type=jnp.float32)
    m = jnp.max(logits, axis=1, keepdims=True)
    ex = jnp.exp(logits - m)
    p = ex / jnp.sum(ex, axis=1, keepdims=True)
    iota = lax.broadcasted_iota(jnp.int32, p.shape, 1)
    m1 = jnp.max(p, axis=1, keepdims=True)
    i1 = jnp.min(jnp.where(p == m1, iota, E), axis=1, keepdims=True)
    pm = jnp.where(iota == i1, _NEG, p)
    m2 = jnp.max(pm, axis=1, keepdims=True)
    i2 = jnp.min(jnp.where(pm == m2, iota, E), axis=1, keepdims=True)
    i1_ref[...] = i1
    i2_ref[...] = i2
    w1_ref[...] = m1
    w2_ref[...] = m2
    sgw = sgw_ref[...]
    sg = lax.dot_general(x, sgw, (((1,), (1,)), ((), ())),
                         preferred_element_type=jnp.float32)
    sig_ref[...] = _sigmoid(sg)


def _dispatch_body(ids_hbm, xb_hbm, poslin_hbm, te_hbm, xperm_hbm,
                   ids_v, pos_v, te_v, sbuf_v, shpos_v,
                   idx0_v, idx1_v, rows0_v, rows1_v, ls0, ls1, ss0, ss1):
    sid = lax.axis_index("s")
    cid = lax.axis_index("c")

    @pl.when(sid == 0)
    def _():
        pltpu.sync_copy(ids_hbm, ids_v)
        lane = lax.broadcasted_iota(jnp.int32, (L,), 0)
        zero16 = jnp.zeros((L,), jnp.int32)

        def cnt(s, cs):
            v = ids_v[pl.ds(s * L, L)]
            return tuple(c + jnp.where(v == e, 1, 0)
                         for e, c in enumerate(cs))

        cs = lax.fori_loop(0, SCH, cnt, (zero16,) * E)

        sbuf_v[pl.ds(0, L)] = zero16
        po = jnp.int32(0)
        bases = []
        incls = []
        for e in range(E):
            sbuf_v[pl.ds(L, L)] = cs[e]
            pref = zero16
            for k in range(1, L):
                pref = pref + sbuf_v[pl.ds(L - k, L)]
            tot = (pref + cs[e])[L - 1]
            bases.append(pref + po)
            po = po + ((tot + T - 1) // T) * T
            incls.append(po)
        for b in range(2):
            tstart = (lane + L * b) * T
            te = zero16
            for e in range(E):
                te = te + jnp.where(incls[e] <= tstart, 1, 0)
            te_v[pl.ds(L * b, L)] = te

        def place(s, curs):
            v = ids_v[pl.ds(s * L, L)]
            pos = zero16
            out = []
            for e in range(E):
                msk = v == e
                pos = jnp.where(msk, curs[e], pos)
                out.append(curs[e] + jnp.where(msk, 1, 0))
            pos_v[pl.ds(s * L, L)] = pos
            return tuple(out)

        lax.fori_loop(0, SCH, place, tuple(bases))
        pltpu.sync_copy(pos_v, shpos_v)

        @pl.when(cid == 0)
        def _():
            pltpu.sync_copy(pos_v, poslin_hbm)
            pltpu.sync_copy(te_v, te_hbm)

    plsc.subcore_barrier()

    w = _wid()
    tw = jnp.where(w >= L, w - L, w)
    nc = 128 // XC
    idxs = (idx0_v, idx1_v)
    bufs = (rows0_v, rows1_v)
    lsems = (ls0, ls1)
    ssems = (ss0, ss1)
    loads = [None, None]
    scats = [None, None]
    pltpu.sync_copy(shpos_v.at[pl.ds(w * 128, XC)], idx0_v)
    loads[0] = pltpu.async_copy(xb_hbm.at[pl.ds(tw * 128, XC)], rows0_v, ls0)
    for c in range(nc):
        b = c % 2
        nb = (c + 1) % 2
        if c + 1 < nc:
            if scats[nb] is not None:
                scats[nb].wait()
            pltpu.sync_copy(
                shpos_v.at[pl.ds(w * 128 + (c + 1) * XC, XC)], idxs[nb])
            loads[nb] = pltpu.async_copy(
                xb_hbm.at[pl.ds(tw * 128 + (c + 1) * XC, XC)],
                bufs[nb], lsems[nb])
        loads[b].wait()
        scats[b] = pltpu.async_copy(bufs[b], xperm_hbm.at[idxs[b]],
                                    ssems[b])
    scats[0].wait()
    scats[1].wait()


def _grouped_body(te_ref, x_ref, w13g_ref, w13u_ref, w2_ref, out_ref):
    t = pl.program_id(0)

    @pl.when(te_ref[t] < E)
    def _():
        xb = x_ref[...].astype(jnp.bfloat16)
        wg = w13g_ref[0].astype(jnp.bfloat16)
        wu = w13u_ref[0].astype(jnp.bfloat16)
        g = lax.dot_general(xb, wg, (((1,), (1,)), ((), ())),
                            preferred_element_type=jnp.float32)
        u = lax.dot_general(xb, wu, (((1,), (1,)), ((), ())),
                            preferred_element_type=jnp.float32)
        h = (g * _sigmoid(g) * u).astype(jnp.bfloat16)
        w2 = w2_ref[0].astype(jnp.bfloat16)
        out_ref[...] = lax.dot_general(h, w2, (((1,), (1,)), ((), ())),
                                       preferred_element_type=jnp.float32)


def _ygather_body(yw_hbm, pos_hbm, y1_hbm, y2_hbm, idx1_v, idx2_v,
                  rows0_v, rows1_v, sem1, sem2):
    w = _wid()
    nc = 64 // YC
    pltpu.sync_copy(pos_hbm.at[pl.ds(w * 64, 64)], idx1_v)
    pltpu.sync_copy(pos_hbm.at[pl.ds(M + w * 64, 64)], idx2_v)
    for c in range(nc):
        tbase = w * 64 + c * YC
        g1 = pltpu.async_copy(yw_hbm.at[idx1_v.at[pl.ds(c * YC, YC)]],
                              rows0_v, sem1)
        g2 = pltpu.async_copy(yw_hbm.at[idx2_v.at[pl.ds(c * YC, YC)]],
                              rows1_v, sem2)
        g1.wait()
        pltpu.sync_copy(rows0_v, y1_hbm.at[pl.ds(tbase, YC)])
        g2.wait()
        pltpu.sync_copy(rows1_v, y2_hbm.at[pl.ds(tbase, YC)])


def _final_body(sh_ref, sig_ref, y1_ref, y2_ref, w1_ref, w2_ref, out_ref):
    out_ref[...] = (sh_ref[...] * sig_ref[...]
                    + w1_ref[...] * y1_ref[...] + w2_ref[...] * y2_ref[...])


def _shared_body(xb_ref, wg_ref, wu_ref, wd_ref, out_ref):
    xb = xb_ref[...]
    wg = wg_ref[...].astype(jnp.bfloat16)
    wu = wu_ref[...].astype(jnp.bfloat16)
    g = lax.dot_general(xb, wg, (((1,), (1,)), ((), ())),
                        preferred_element_type=jnp.float32)
    u = lax.dot_general(xb, wu, (((1,), (1,)), ((), ())),
                        preferred_element_type=jnp.float32)
    h = (g * _sigmoid(g) * u).astype(jnp.bfloat16)
    wd = wd_ref[...].astype(jnp.bfloat16)
    y = lax.dot_general(h, wd, (((1,), (1,)), ((), ())),
                        preferred_element_type=jnp.float32)
    j = pl.program_id(0)

    @pl.when(j == 0)
    def _():
        out_ref[...] = y

    @pl.when(j > 0)
    def _():
        out_ref[...] += y


@functools.partial(jax.jit, static_argnames=("interpret",))
def _run(x32, gate_w, shared_expert_gate_w, shared_gate_up_w, shared_down_w,
         w13_stacked, w2_stacked, interpret=False):
    xb = x32.astype(jnp.bfloat16)

    i1, i2, w1, w2c, sig = pl.pallas_call(
        _router_body,
        out_shape=(jax.ShapeDtypeStruct((M, 1), jnp.int32),
                   jax.ShapeDtypeStruct((M, 1), jnp.int32),
                   jax.ShapeDtypeStruct((M, 1), jnp.float32),
                   jax.ShapeDtypeStruct((M, 1), jnp.float32),
                   jax.ShapeDtypeStruct((M, 1), jnp.float32)),
        interpret=interpret,
    )(x32, gate_w, shared_expert_gate_w)

    ids_km = jnp.concatenate([i1, i2], axis=0).reshape(A)

    sc_mesh = plsc.VectorSubcoreMesh(core_axis_name="c", subcore_axis_name="s")

    poslin, te, xperm = pl.kernel(
        _dispatch_body,
        out_type=(jax.ShapeDtypeStruct((A,), jnp.int32),
                  jax.ShapeDtypeStruct((NW,), jnp.int32),
                  jax.ShapeDtypeStruct((NP, H), jnp.float32)),
        mesh=sc_mesh,
        scratch_types=[pltpu.VMEM((A,), jnp.int32),
                       pltpu.VMEM((A,), jnp.int32),
                       pltpu.VMEM((NW,), jnp.int32),
                       pltpu.VMEM((2 * L,), jnp.int32),
                       pltpu.VMEM_SHARED((A,), jnp.int32),
                       pltpu.VMEM((XC,), jnp.int32),
                       pltpu.VMEM((XC,), jnp.int32),
                       pltpu.VMEM((XC, H), jnp.float32),
                       pltpu.VMEM((XC, H), jnp.float32),
                       pltpu.SemaphoreType.DMA,
                       pltpu.SemaphoreType.DMA,
                       pltpu.SemaphoreType.DMA,
                       pltpu.SemaphoreType.DMA],
    )(ids_km, x32)

    yw = pl.pallas_call(
        _grouped_body,
        grid_spec=pltpu.PrefetchScalarGridSpec(
            num_scalar_prefetch=1,
            grid=(NT,),
            in_specs=[
                pl.BlockSpec((T, H), lambda t, te_r: (t, 0)),
                pl.BlockSpec((1, I, H),
                             lambda t, te_r: (jnp.minimum(te_r[t], E - 1),
                                              0, 0)),
                pl.BlockSpec((1, I, H),
                             lambda t, te_r: (jnp.minimum(te_r[t], E - 1),
                                              1, 0)),
                pl.BlockSpec((1, H, I),
                             lambda t, te_r: (jnp.minimum(te_r[t], E - 1),
                                              0, 0)),
            ],
            out_specs=pl.BlockSpec((T, H), lambda t, te_r: (t, 0)),
        ),
        out_shape=jax.ShapeDtypeStruct((NP, H), jnp.float32),
        compiler_params=pltpu.CompilerParams(
            vmem_limit_bytes=63 * 1024 * 1024),
        interpret=interpret,
    )(te, xperm, w13_stacked, w13_stacked, w2_stacked)

    y1, y2 = pl.kernel(
        _ygather_body,
        out_type=(jax.ShapeDtypeStruct((M, H), jnp.float32),
                  jax.ShapeDtypeStruct((M, H), jnp.float32)),
        mesh=sc_mesh,
        scratch_types=[pltpu.VMEM((64,), jnp.int32),
                       pltpu.VMEM((64,), jnp.int32),
                       pltpu.VMEM((YC, H), jnp.float32),
                       pltpu.VMEM((YC, H), jnp.float32),
                       pltpu.SemaphoreType.DMA,
                       pltpu.SemaphoreType.DMA],
    )(yw, poslin)

    sh = pl.pallas_call(
        _shared_body,
        grid=(NJ,),
        in_specs=[
            pl.BlockSpec((M, H), lambda j: (0, 0)),
            pl.BlockSpec((BJ, H), lambda j: (j, 0)),
            pl.BlockSpec((BJ, H), lambda j: (j + NJ, 0)),
            pl.BlockSpec((H, BJ), lambda j: (0, j)),
        ],
        out_specs=pl.BlockSpec((M, H), lambda j: (0, 0)),
        out_shape=jax.ShapeDtypeStruct((M, H), jnp.float32),
        interpret=interpret,
    )(xb, shared_gate_up_w, shared_gate_up_w, shared_down_w)

    out = pl.pallas_call(
        _final_body,
        out_shape=jax.ShapeDtypeStruct((M, H), jnp.float32),
        interpret=interpret,
    )(sh, sig, y1, y2, w1, w2c)
    return out


def kernel(hidden_states, gate_w, shared_expert_gate_w, shared_gate_up_w,
           shared_down_w, w13_stacked, w2_stacked):
    orig_shape = hidden_states.shape
    x32 = hidden_states.reshape(-1, H).astype(jnp.float32)
    out = _run(x32, gate_w, shared_expert_gate_w, shared_gate_up_w,
               shared_down_w, w13_stacked, w2_stacked)
    return out.astype(hidden_states.dtype).reshape(orig_shape)


# speedup vs baseline: 1.1172x; 1.0020x over previous
import functools

import jax
import jax.numpy as jnp
from jax import lax
from jax.experimental import pallas as pl
from jax.experimental.pallas import tpu as pltpu
from jax.experimental.pallas import tpu_sc as plsc

H = 1024
E = 8
TOPK = 2
I = 1408
ISH = 5632

M = 2048
A = M * TOPK
T = 640
NT = 15
NP = NT * T
BJ = 512
NJ = ISH // BJ

NW = 32
L = 16
SCH = A // L
XC = 32
YC = 32

_NEG = -1e30


def _sigmoid(x):
    return 1.0 / (1.0 + jnp.exp(-x))


def _wid():
    return lax.axis_index("s") * 2 + lax.axis_index("c")


def _router_body(x_ref, gw_ref, sgw_ref, i1_ref, i2_ref, w1_ref, w2_ref,
                 sig_ref):
    x = x_ref[...]
    gw = gw_ref[...]
    logits = lax.dot_general(x, gw, (((1,), (1,)), ((), ())),
                             preferred_element_type=jnp.float32)
    m = jnp.max(logits, axis=1, keepdims=True)
    ex = jnp.exp(logits - m)
    p = ex / jnp.sum(ex, axis=1, keepdims=True)
    iota = lax.broadcasted_iota(jnp.int32, p.shape, 1)
    m1 = jnp.max(p, axis=1, keepdims=True)
    i1 = jnp.min(jnp.where(p == m1, iota, E), axis=1, keepdims=True)
    pm = jnp.where(iota == i1, _NEG, p)
    m2 = jnp.max(pm, axis=1, keepdims=True)
    i2 = jnp.min(jnp.where(pm == m2, iota, E), axis=1, keepdims=True)
    i1_ref[...] = i1
    i2_ref[...] = i2
    w1_ref[...] = m1
    w2_ref[...] = m2
    sgw = sgw_ref[...]
    sg = lax.dot_general(x, sgw, (((1,), (1,)), ((), ())),
                         preferred_element_type=jnp.float32)
    sig_ref[...] = _sigmoid(sg)


def _dispatch_body(ids_hbm, xb_hbm, poslin_hbm, te_hbm, xperm_hbm,
                   ids_v, pos_v, te_v, sbuf_v, shpos_v,
                   idx0_v, idx1_v, rows0_v, rows1_v, ls0, ls1, ss0, ss1):
    sid = lax.axis_index("s")
    cid = lax.axis_index("c")

    @pl.when(sid == 0)
    def _():
        pltpu.sync_copy(ids_hbm, ids_v)
        lane = lax.broadcasted_iota(jnp.int32, (L,), 0)
        zero16 = jnp.zeros((L,), jnp.int32)

        def cnt(s, cs):
            v = ids_v[pl.ds(s * L, L)]
            return tuple(c + jnp.where(v == e, 1, 0)
                         for e, c in enumerate(cs))

        cs = lax.fori_loop(0, SCH, cnt, (zero16,) * E)

        sbuf_v[pl.ds(0, L)] = zero16
        po = jnp.int32(0)
        bases = []
        incls = []
        for e in range(E):
            sbuf_v[pl.ds(L, L)] = cs[e]
            pref = zero16
            for k in range(1, L):
                pref = pref + sbuf_v[pl.ds(L - k, L)]
            tot = (pref + cs[e])[L - 1]
            bases.append(pref + po)
            po = po + ((tot + T - 1) // T) * T
            incls.append(po)
        for b in range(2):
            tstart = (lane + L * b) * T
            te = zero16
            for e in range(E):
                te = te + jnp.where(incls[e] <= tstart, 1, 0)
            te_v[pl.ds(L * b, L)] = te

        def place(s, curs):
            v = ids_v[pl.ds(s * L, L)]
            pos = zero16
            out = []
            for e in range(E):
                msk = v == e
                pos = jnp.where(msk, curs[e], pos)
                out.append(curs[e] + jnp.where(msk, 1, 0))
            pos_v[pl.ds(s * L, L)] = pos
            return tuple(out)

        lax.fori_loop(0, SCH, place, tuple(bases))
        pltpu.sync_copy(pos_v, shpos_v)

        @pl.when(cid == 0)
        def _():
            pltpu.sync_copy(pos_v, poslin_hbm)
            pltpu.sync_copy(te_v, te_hbm)

    plsc.subcore_barrier()

    w = _wid()
    tw = jnp.where(w >= L, w - L, w)
    nc = 128 // XC
    idxs = (idx0_v, idx1_v)
    bufs = (rows0_v, rows1_v)
    lsems = (ls0, ls1)
    ssems = (ss0, ss1)
    loads = [None, None]
    scats = [None, None]
    pltpu.sync_copy(shpos_v.at[pl.ds(w * 128, XC)], idx0_v)
    loads[0] = pltpu.async_copy(xb_hbm.at[pl.ds(tw * 128, XC)], rows0_v, ls0)
    for c in range(nc):
        b = c % 2
        nb = (c + 1) % 2
        if c + 1 < nc:
            if scats[nb] is not None:
                scats[nb].wait()
            pltpu.sync_copy(
                shpos_v.at[pl.ds(w * 128 + (c + 1) * XC, XC)], idxs[nb])
            loads[nb] = pltpu.async_copy(
                xb_hbm.at[pl.ds(tw * 128 + (c + 1) * XC, XC)],
                bufs[nb], lsems[nb])
        loads[b].wait()
        scats[b] = pltpu.async_copy(bufs[b], xperm_hbm.at[idxs[b]],
                                    ssems[b])
    scats[0].wait()
    scats[1].wait()


def _grouped_body(te_ref, x_ref, w13g_ref, w13u_ref, w2_ref, out_ref):
    t = pl.program_id(0)

    @pl.when(te_ref[t] < E)
    def _():
        xb = x_ref[...].astype(jnp.bfloat16)
        wg = w13g_ref[0].astype(jnp.bfloat16)
        wu = w13u_ref[0].astype(jnp.bfloat16)
        g = lax.dot_general(xb, wg, (((1,), (1,)), ((), ())),
                            preferred_element_type=jnp.float32)
        u = lax.dot_general(xb, wu, (((1,), (1,)), ((), ())),
                            preferred_element_type=jnp.float32)
        h = (g * _sigmoid(g) * u).astype(jnp.bfloat16)
        w2 = w2_ref[0].astype(jnp.bfloat16)
        out_ref[...] = lax.dot_general(h, w2, (((1,), (1,)), ((), ())),
                                       preferred_element_type=jnp.float32)


def _ygather_body(yw_hbm, pos_hbm, y1_hbm, y2_hbm, idx1_v, idx2_v,
                  rows0_v, rows1_v, sem1, sem2):
    w = _wid()
    nc = 64 // YC
    pltpu.sync_copy(pos_hbm.at[pl.ds(w * 64, 64)], idx1_v)
    pltpu.sync_copy(pos_hbm.at[pl.ds(M + w * 64, 64)], idx2_v)
    for c in range(nc):
        tbase = w * 64 + c * YC
        g1 = pltpu.async_copy(yw_hbm.at[idx1_v.at[pl.ds(c * YC, YC)]],
                              rows0_v, sem1)
        g2 = pltpu.async_copy(yw_hbm.at[idx2_v.at[pl.ds(c * YC, YC)]],
                              rows1_v, sem2)
        g1.wait()
        pltpu.sync_copy(rows0_v, y1_hbm.at[pl.ds(tbase, YC)])
        g2.wait()
        pltpu.sync_copy(rows1_v, y2_hbm.at[pl.ds(tbase, YC)])


def _final_body(sh_ref, sig_ref, y1_ref, y2_ref, w1_ref, w2_ref, out_ref):
    out_ref[...] = (sh_ref[...] * sig_ref[...]
                    + w1_ref[...] * y1_ref[...] + w2_ref[...] * y2_ref[...])


def _shared_body(xb_ref, wg_ref, wu_ref, wd_ref, out_ref):
    xb = xb_ref[...]
    wg = wg_ref[...].astype(jnp.bfloat16)
    wu = wu_ref[...].astype(jnp.bfloat16)
    g = lax.dot_general(xb, wg, (((1,), (1,)), ((), ())),
                        preferred_element_type=jnp.float32)
    u = lax.dot_general(xb, wu, (((1,), (1,)), ((), ())),
                        preferred_element_type=jnp.float32)
    h = (g * _sigmoid(g) * u).astype(jnp.bfloat16)
    wd = wd_ref[...].astype(jnp.bfloat16)
    y = lax.dot_general(h, wd, (((1,), (1,)), ((), ())),
                        preferred_element_type=jnp.float32)
    j = pl.program_id(0)

    @pl.when(j == 0)
    def _():
        out_ref[...] = y

    @pl.when(j > 0)
    def _():
        out_ref[...] += y


@functools.partial(jax.jit, static_argnames=("interpret",))
def _run(x32, gate_w, shared_expert_gate_w, shared_gate_up_w, shared_down_w,
         w13_stacked, w2_stacked, interpret=False):
    xb = x32.astype(jnp.bfloat16)

    i1, i2, w1, w2c, sig = pl.pallas_call(
        _router_body,
        out_shape=(jax.ShapeDtypeStruct((M, 1), jnp.int32),
                   jax.ShapeDtypeStruct((M, 1), jnp.int32),
                   jax.ShapeDtypeStruct((M, 1), jnp.float32),
                   jax.ShapeDtypeStruct((M, 1), jnp.float32),
                   jax.ShapeDtypeStruct((M, 1), jnp.float32)),
        interpret=interpret,
    )(x32, gate_w, shared_expert_gate_w)

    ids_km = jnp.concatenate([i1, i2], axis=0).reshape(A)

    sc_mesh = plsc.VectorSubcoreMesh(core_axis_name="c", subcore_axis_name="s")

    poslin, te, xperm = pl.kernel(
        _dispatch_body,
        out_type=(jax.ShapeDtypeStruct((A,), jnp.int32),
                  jax.ShapeDtypeStruct((NW,), jnp.int32),
                  jax.ShapeDtypeStruct((NP, H), jnp.float32)),
        mesh=sc_mesh,
        scratch_types=[pltpu.VMEM((A,), jnp.int32),
                       pltpu.VMEM((A,), jnp.int32),
                       pltpu.VMEM((NW,), jnp.int32),
                       pltpu.VMEM((2 * L,), jnp.int32),
                       pltpu.VMEM_SHARED((A,), jnp.int32),
                       pltpu.VMEM((XC,), jnp.int32),
                       pltpu.VMEM((XC,), jnp.int32),
                       pltpu.VMEM((XC, H), jnp.float32),
                       pltpu.VMEM((XC, H), jnp.float32),
                       pltpu.SemaphoreType.DMA,
                       pltpu.SemaphoreType.DMA,
                       pltpu.SemaphoreType.DMA,
                       pltpu.SemaphoreType.DMA],
    )(ids_km, x32)

    yw = pl.pallas_call(
        _grouped_body,
        grid_spec=pltpu.PrefetchScalarGridSpec(
            num_scalar_prefetch=1,
            grid=(NT,),
            in_specs=[
                pl.BlockSpec((T, H), lambda t, te_r: (t, 0)),
                pl.BlockSpec((1, I, H),
                             lambda t, te_r: (jnp.minimum(te_r[t], E - 1),
                                              0, 0)),
                pl.BlockSpec((1, I, H),
                             lambda t, te_r: (jnp.minimum(te_r[t], E - 1),
                                              1, 0)),
                pl.BlockSpec((1, H, I),
                             lambda t, te_r: (jnp.minimum(te_r[t], E - 1),
                                              0, 0)),
            ],
            out_specs=pl.BlockSpec((T, H), lambda t, te_r: (t, 0)),
        ),
        out_shape=jax.ShapeDtypeStruct((NP, H), jnp.float32),
        compiler_params=pltpu.CompilerParams(
            vmem_limit_bytes=63 * 1024 * 1024),
        interpret=interpret,
    )(te, xperm, w13_stacked, w13_stacked, w2_stacked)

    y1, y2 = pl.kernel(
        _ygather_body,
        out_type=(jax.ShapeDtypeStruct((M, H), jnp.float32),
                  jax.ShapeDtypeStruct((M, H), jnp.float32)),
        mesh=sc_mesh,
        scratch_types=[pltpu.VMEM((64,), jnp.int32),
                       pltpu.VMEM((64,), jnp.int32),
                       pltpu.VMEM((YC, H), jnp.float32),
                       pltpu.VMEM((YC, H), jnp.float32),
                       pltpu.SemaphoreType.DMA,
                       pltpu.SemaphoreType.DMA],
    )(yw, poslin)

    sh = pl.pallas_call(
        _shared_body,
        grid=(NJ,),
        in_specs=[
            pl.BlockSpec((M, H), lambda j: (0, 0)),
            pl.BlockSpec((BJ, H), lambda j: (j, 0)),
            pl.BlockSpec((BJ, H), lambda j: (j + NJ, 0)),
            pl.BlockSpec((H, BJ), lambda j: (0, j)),
        ],
        out_specs=pl.BlockSpec((M, H), lambda j: (0, 0)),
        out_shape=jax.ShapeDtypeStruct((M, H), jnp.float32),
        interpret=interpret,
    )(xb, shared_gate_up_w, shared_gate_up_w, shared_down_w)

    out = pl.pallas_call(
        _final_body,
        out_shape=jax.ShapeDtypeStruct((M, H), jnp.float32),
        interpret=interpret,
    )(sh, sig, y1, y2, w1, w2c)
    return out


def kernel(hidden_states, gate_w, shared_expert_gate_w, shared_gate_up_w,
           shared_down_w, w13_stacked, w2_stacked):
    orig_shape = hidden_states.shape
    x32 = hidden_states.reshape(-1, H).astype(jnp.float32)
    out = _run(x32, gate_w, shared_expert_gate_w, shared_gate_up_w,
               shared_down_w, w13_stacked, w2_stacked)
    return out.astype(hidden_states.dtype).reshape(orig_shape)
